# initial kernel scaffold (unmeasured)
import jax
import jax.numpy as jnp
from jax import lax
from jax.experimental import pallas as pl
from jax.experimental.pallas import tpu as pltpu

N_DEV = 4
B_LOC = 2
SQ = 128
SKV = 128
D_MODEL = 512
H_LOC = 4
DH = 64
DH_LOC = H_LOC * DH


def kernel(x, Wq, K_ext, V_ext, Wo):
    def body(x_ref, wq_ref, k_ref, v_ref, wo_ref, out_ref,
             wq_all, wo_all, k_bufs, v_bufs,
             send_sems, recv_sems, kv_sems):
        my = lax.axis_index("i")

        def group_of(slot):
            return (my - slot) % N_DEV

        kv_copies = []
        for s in range(N_DEV):
            g = group_of(s)
            ck = pltpu.make_async_copy(
                k_ref.at[pl.ds(my * B_LOC, B_LOC), :, pl.ds(g * H_LOC, H_LOC), :],
                k_bufs.at[s], kv_sems.at[0, s])
            cv = pltpu.make_async_copy(
                v_ref.at[pl.ds(my * B_LOC, B_LOC), :, pl.ds(g * H_LOC, H_LOC), :],
                v_bufs.at[s], kv_sems.at[1, s])
            ck.start()
            cv.start()
            kv_copies.append((ck, cv))

        barrier = pltpu.get_barrier_semaphore()
        for off in range(1, N_DEV):
            pl.semaphore_signal(
                barrier, inc=1,
                device_id=((my + off) % N_DEV,),
                device_id_type=pl.DeviceIdType.MESH,
            )
        pl.semaphore_wait(barrier, N_DEV - 1)

        wq_all[0] = wq_ref[...].astype(jnp.bfloat16)
        wo_all[0] = wo_ref[...].astype(jnp.bfloat16)

        rdmas = {}
        for off in range(1, N_DEV):
            for t, buf in ((0, wq_all), (1, wo_all)):
                r = pltpu.make_async_remote_copy(
                    src_ref=buf.at[0],
                    dst_ref=buf.at[off],
                    send_sem=send_sems.at[t, off],
                    recv_sem=recv_sems.at[t, off],
                    device_id=((my + off) % N_DEV,),
                    device_id_type=pl.DeviceIdType.MESH,
                )
                r.start()
                rdmas[(t, off)] = r

        x2 = x_ref[...].reshape(B_LOC * SQ, D_MODEL).astype(jnp.bfloat16)

        def group_out(slot):
            wq_g = wq_all[slot]
            wo_g = wo_all[slot]
            qg = jnp.dot(x2, wq_g, preferred_element_type=jnp.float32)
            kg = k_bufs[slot].astype(jnp.bfloat16)
            vg = v_bufs[slot].astype(jnp.bfloat16)
            acc = None
            for h in range(H_LOC):
                q = (qg[:, h * DH:(h + 1) * DH]
                     .astype(jnp.bfloat16).reshape(B_LOC, SQ, DH))
                k = kg[:, :, h, :]
                v = vg[:, :, h, :]
                scores = lax.dot_general(
                    q, k, (((2,), (2,)), ((0,), (0,))),
                    preferred_element_type=jnp.float32) * 0.125
                qi = lax.broadcasted_iota(jnp.int32, scores.shape, 1)
                kj = lax.broadcasted_iota(jnp.int32, scores.shape, 2)
                scores = jnp.where((qi < 64) & (kj >= 64),
                                   jnp.float32(-1e9), scores)
                m = jnp.max(scores, axis=-1, keepdims=True)
                w = jnp.exp(scores - m)
                w = w / jnp.sum(w, axis=-1, keepdims=True)
                ctx = lax.dot_general(
                    w.astype(jnp.bfloat16), v, (((2,), (1,)), ((0,), (0,))),
                    preferred_element_type=jnp.float32)
                part = jnp.dot(
                    ctx.reshape(B_LOC * SQ, DH).astype(jnp.bfloat16),
                    wo_g[h * DH:(h + 1) * DH, :],
                    preferred_element_type=jnp.float32)
                acc = part if acc is None else acc + part
            return acc

        kv_copies[0][0].wait()
        kv_copies[0][1].wait()
        out = group_out(0)
        for off in (1, 3, 2):
            rdmas[(0, off)].wait_recv()
            rdmas[(1, off)].wait_recv()
            kv_copies[off][0].wait()
            kv_copies[off][1].wait()
            out = out + group_out(off)

        for r in rdmas.values():
            r.wait_send()

        out_ref[...] = out.reshape(B_LOC, SQ, D_MODEL)

    return pl.pallas_call(
        body,
        out_shape=jax.ShapeDtypeStruct((B_LOC, SQ, D_MODEL), jnp.float32),
        in_specs=[
            pl.BlockSpec(memory_space=pltpu.VMEM),
            pl.BlockSpec(memory_space=pltpu.VMEM),
            pl.BlockSpec(memory_space=pltpu.ANY),
            pl.BlockSpec(memory_space=pltpu.ANY),
            pl.BlockSpec(memory_space=pltpu.VMEM),
        ],
        out_specs=pl.BlockSpec(memory_space=pltpu.VMEM),
        scratch_shapes=[
            pltpu.VMEM((N_DEV, D_MODEL, DH_LOC), jnp.bfloat16),
            pltpu.VMEM((N_DEV, DH_LOC, D_MODEL), jnp.bfloat16),
            pltpu.VMEM((N_DEV, B_LOC, SKV, H_LOC, DH), jnp.float32),
            pltpu.VMEM((N_DEV, B_LOC, SKV, H_LOC, DH), jnp.float32),
            pltpu.SemaphoreType.DMA((2, N_DEV)),
            pltpu.SemaphoreType.DMA((2, N_DEV)),
            pltpu.SemaphoreType.DMA((2, N_DEV)),
        ],
        compiler_params=pltpu.CompilerParams(collective_id=0),
    )(x, Wq, K_ext, V_ext, Wo)


# baseline (device time: 32542 ns/iter reference)
import jax
import jax.numpy as jnp
from jax import lax
from jax.experimental import pallas as pl
from jax.experimental.pallas import tpu as pltpu

N_DEV = 4
B_LOC = 2
SQ = 128
SKV = 128
D_MODEL = 512
H_LOC = 4
DH = 64
DH_LOC = H_LOC * DH


def kernel(x, Wq, K_ext, V_ext, Wo):
    def body(x_ref, wq_ref, k_ref, v_ref, wo_ref, out_ref,
             wq_all, wo_all, k_bufs, v_bufs,
             send_sems, recv_sems, kv_sems):
        my = lax.axis_index("i")

        def group_of(slot):
            return (my - slot) % N_DEV

        kv_copies = []
        for s in range(N_DEV):
            g = group_of(s)
            ck = pltpu.make_async_copy(
                k_ref.at[pl.ds(my * B_LOC, B_LOC), :, pl.ds(g * H_LOC, H_LOC), :],
                k_bufs.at[s], kv_sems.at[0, s])
            cv = pltpu.make_async_copy(
                v_ref.at[pl.ds(my * B_LOC, B_LOC), :, pl.ds(g * H_LOC, H_LOC), :],
                v_bufs.at[s], kv_sems.at[1, s])
            ck.start()
            cv.start()
            kv_copies.append((ck, cv))

        barrier = pltpu.get_barrier_semaphore()
        for off in range(1, N_DEV):
            pl.semaphore_signal(
                barrier, inc=1,
                device_id=((my + off) % N_DEV,),
                device_id_type=pl.DeviceIdType.MESH,
            )
        pl.semaphore_wait(barrier, N_DEV - 1)

        wq_all[0] = wq_ref[...].astype(jnp.bfloat16)
        wo_all[0] = wo_ref[...].astype(jnp.bfloat16)

        rdmas = {}
        for off in range(1, N_DEV):
            for t, buf in ((0, wq_all), (1, wo_all)):
                r = pltpu.make_async_remote_copy(
                    src_ref=buf.at[0],
                    dst_ref=buf.at[off],
                    send_sem=send_sems.at[t, off],
                    recv_sem=recv_sems.at[t, off],
                    device_id=((my + off) % N_DEV,),
                    device_id_type=pl.DeviceIdType.MESH,
                )
                r.start()
                rdmas[(t, off)] = r

        x2 = x_ref[...].reshape(B_LOC * SQ, D_MODEL).astype(jnp.bfloat16)

        def group_out(slot):
            wq_g = wq_all[slot]
            wo_g = wo_all[slot]
            qg = jnp.dot(x2, wq_g, preferred_element_type=jnp.float32)
            kg = k_bufs[slot].astype(jnp.bfloat16)
            vg = v_bufs[slot].astype(jnp.bfloat16)
            acc = None
            for h in range(H_LOC):
                q = (qg[:, h * DH:(h + 1) * DH]
                     .astype(jnp.bfloat16).reshape(B_LOC, SQ, DH))
                k = kg[:, :, h, :]
                v = vg[:, :, h, :]
                scores = lax.dot_general(
                    q, k, (((2,), (2,)), ((0,), (0,))),
                    preferred_element_type=jnp.float32) * 0.125
                qi = lax.broadcasted_iota(jnp.int32, scores.shape, 1)
                kj = lax.broadcasted_iota(jnp.int32, scores.shape, 2)
                scores = jnp.where((qi < 64) & (kj >= 64),
                                   jnp.float32(-1e9), scores)
                m = jnp.max(scores, axis=-1, keepdims=True)
                w = jnp.exp(scores - m)
                w = w / jnp.sum(w, axis=-1, keepdims=True)
                ctx = lax.dot_general(
                    w.astype(jnp.bfloat16), v, (((2,), (1,)), ((0,), (0,))),
                    preferred_element_type=jnp.float32)
                part = jnp.dot(
                    ctx.reshape(B_LOC * SQ, DH).astype(jnp.bfloat16),
                    wo_g[h * DH:(h + 1) * DH, :],
                    preferred_element_type=jnp.float32)
                acc = part if acc is None else acc + part
            return acc

        kv_copies[0][0].wait()
        kv_copies[0][1].wait()
        out = group_out(0)
        for off in (1, 3, 2):
            rdmas[(0, off)].wait_recv()
            rdmas[(1, off)].wait_recv()
            kv_copies[off][0].wait()
            kv_copies[off][1].wait()
            out = out + group_out(off)

        for r in rdmas.values():
            r.wait_send()

        out_ref[...] = out.reshape(B_LOC, SQ, D_MODEL)

    return pl.pallas_call(
        body,
        out_shape=jax.ShapeDtypeStruct((B_LOC, SQ, D_MODEL), jnp.float32),
        in_specs=[
            pl.BlockSpec(memory_space=pltpu.VMEM),
            pl.BlockSpec(memory_space=pltpu.VMEM),
            pl.BlockSpec(memory_space=pl.ANY),
            pl.BlockSpec(memory_space=pl.ANY),
            pl.BlockSpec(memory_space=pltpu.VMEM),
        ],
        out_specs=pl.BlockSpec(memory_space=pltpu.VMEM),
        scratch_shapes=[
            pltpu.VMEM((N_DEV, D_MODEL, DH_LOC), jnp.bfloat16),
            pltpu.VMEM((N_DEV, DH_LOC, D_MODEL), jnp.bfloat16),
            pltpu.VMEM((N_DEV, B_LOC, SKV, H_LOC, DH), jnp.float32),
            pltpu.VMEM((N_DEV, B_LOC, SKV, H_LOC, DH), jnp.float32),
            pltpu.SemaphoreType.DMA((2, N_DEV)),
            pltpu.SemaphoreType.DMA((2, N_DEV)),
            pltpu.SemaphoreType.DMA((2, N_DEV)),
        ],
        compiler_params=pltpu.CompilerParams(collective_id=0),
    )(x, Wq, K_ext, V_ext, Wo)
